# Initial kernel scaffold; baseline (speedup 1.0000x reference)
import jax
import jax.numpy as jnp
from jax.experimental import pallas as pl


def kernel(prefix, table):
    raise NotImplementedError("write your pallas kernel here")


# speedup vs baseline: 1.7948x; 1.7948x over previous
import functools

import jax
import jax.numpy as jnp
from jax import lax
from jax.experimental import pallas as pl
from jax.experimental.pallas import tpu as pltpu
from jax.experimental.pallas import tpu_sc as plsc

_N_LAYERS = 24
_N_HEADS = 32
_HEAD_DIM = 64
_PRE_SEQ_LEN = 64
_BATCH = 4
_ROW_STRIDE = _N_LAYERS * 2 * _N_HEADS
_ROWS_PER_OUT = 2 * _BATCH * _N_HEADS * _PRE_SEQ_LEN
_NW = 32
_ROWS_PER_W = _ROWS_PER_OUT // _NW
_CHUNK = 128
_NCHUNK = _ROWS_PER_W // _CHUNK


def _body(prefix_hbm, table_hbm, *refs):
    outs = refs[:_N_LAYERS]
    pref_v, idx_v, buf, sem = refs[_N_LAYERS:]

    wid = lax.axis_index("s") * 2 + lax.axis_index("c")
    kv = wid // 16
    b = (wid // 4) % 4
    h0 = (wid % 4) * 8
    woff = kv * 32 + h0 - 64
    base_out = wid * _ROWS_PER_W

    pltpu.sync_copy(prefix_hbm.at[b], pref_v)

    for c in range(_NCHUNK):
        for j in range(8):
            r0 = c * _CHUNK + j * 16
            keys = pref_v[pl.ds(r0 % _PRE_SEQ_LEN, 16)]
            idx_v[c, pl.ds(j * 16, 16)] = keys * _ROW_STRIDE + (woff + r0 // 64)

    for i in range(_N_LAYERS):
        out_i = outs[i]

        @pl.loop(0, _NCHUNK)
        def _chunk(c):
            for j in range(8):
                sl = pl.ds(j * 16, 16)
                idx_v[c, sl] = idx_v[c, sl] + 64
            pltpu.async_copy(table_hbm.at[idx_v.at[c]], buf, sem).wait()
            pltpu.sync_copy(buf, out_i.at[pl.ds(base_out + c * _CHUNK, _CHUNK)])


@functools.partial(jax.jit, static_argnames=())
def _sc_gather(prefix, table_r):
    mesh = plsc.VectorSubcoreMesh(core_axis_name="c", subcore_axis_name="s")
    out_type = [jax.ShapeDtypeStruct((_ROWS_PER_OUT, _HEAD_DIM), jnp.float32)
                for _ in range(_N_LAYERS)]
    scratch = [
        pltpu.VMEM((_PRE_SEQ_LEN,), jnp.int32),
        pltpu.VMEM((_NCHUNK, _CHUNK), jnp.int32),
        pltpu.VMEM((_CHUNK, _HEAD_DIM), jnp.float32),
        pltpu.SemaphoreType.DMA,
    ]
    return pl.kernel(
        _body, out_type=out_type, mesh=mesh, scratch_types=scratch,
        compiler_params=pltpu.CompilerParams(use_tc_tiling_on_sc=False),
    )(prefix, table_r)


def kernel(prefix, table):
    table_r = table.reshape(_PRE_SEQ_LEN * _ROW_STRIDE, _HEAD_DIM)
    outs = _sc_gather(prefix, table_r)
    return tuple(
        o.reshape(2, _BATCH, _N_HEADS, _PRE_SEQ_LEN, _HEAD_DIM) for o in outs)


# speedup vs baseline: 2.1913x; 1.2209x over previous
import functools

import jax
import jax.numpy as jnp
from jax import lax
from jax.experimental import pallas as pl
from jax.experimental.pallas import tpu as pltpu
from jax.experimental.pallas import tpu_sc as plsc

_N_LAYERS = 24
_N_HEADS = 32
_HEAD_DIM = 64
_PRE_SEQ_LEN = 64
_BATCH = 4
_ROW_STRIDE = _N_LAYERS * 2 * _N_HEADS
_ROWS_PER_OUT = 2 * _BATCH * _N_HEADS * _PRE_SEQ_LEN
_NW = 32
_ROWS_PER_W = _ROWS_PER_OUT // _NW
_CHUNK = 128
_NCHUNK = _ROWS_PER_W // _CHUNK


def _body(prefix_hbm, table_hbm, *refs):
    outs = refs[:_N_LAYERS]
    pref_v, idx_v, buf, sem = refs[_N_LAYERS:]

    wid = lax.axis_index("s") * 2 + lax.axis_index("c")
    kv = wid // 16
    b = (wid // 4) % 4
    h0 = (wid % 4) * 8
    woff = kv * 32 + h0 - 64
    base_out = wid * _ROWS_PER_W

    pltpu.sync_copy(prefix_hbm.at[b], pref_v)

    for c in range(_NCHUNK):
        for j in range(8):
            r0 = c * _CHUNK + j * 16
            keys = pref_v[pl.ds(r0 % _PRE_SEQ_LEN, 16)]
            idx_v[c, pl.ds(j * 16, 16)] = keys * _ROW_STRIDE + (woff + r0 // 64)

    def _bump_and_issue(nc):
        r = lax.rem(nc, _NCHUNK)
        q = lax.rem(nc, 2)
        for j in range(8):
            sl = pl.ds(j * 16, 16)
            idx_v[r, sl] = idx_v[r, sl] + 64
        pltpu.async_copy(table_hbm.at[idx_v.at[r]], buf.at[q], sem.at[q])

    _bump_and_issue(0)

    for i in range(_N_LAYERS):
        out_i = outs[i]

        @pl.loop(0, _NCHUNK)
        def _chunk(c):
            if i < _N_LAYERS - 1:
                _bump_and_issue(c + 1)
            else:
                @pl.when(c + 1 < _NCHUNK)
                def _():
                    _bump_and_issue(c + 1)
            p = lax.rem(c, 2)
            pltpu.make_async_copy(
                table_hbm.at[idx_v.at[c]], buf.at[p], sem.at[p]).wait()
            pltpu.sync_copy(
                buf.at[p], out_i.at[pl.ds(base_out + c * _CHUNK, _CHUNK)])


@functools.partial(jax.jit, static_argnames=())
def _sc_gather(prefix, table_r):
    mesh = plsc.VectorSubcoreMesh(core_axis_name="c", subcore_axis_name="s")
    out_type = [jax.ShapeDtypeStruct((_ROWS_PER_OUT, _HEAD_DIM), jnp.float32)
                for _ in range(_N_LAYERS)]
    scratch = [
        pltpu.VMEM((_PRE_SEQ_LEN,), jnp.int32),
        pltpu.VMEM((_NCHUNK, _CHUNK), jnp.int32),
        pltpu.VMEM((2, _CHUNK, _HEAD_DIM), jnp.float32),
        pltpu.SemaphoreType.DMA((2,)),
    ]
    return pl.kernel(
        _body, out_type=out_type, mesh=mesh, scratch_types=scratch,
        compiler_params=pltpu.CompilerParams(use_tc_tiling_on_sc=False),
    )(prefix, table_r)


def kernel(prefix, table):
    table_r = table.reshape(_PRE_SEQ_LEN * _ROW_STRIDE, _HEAD_DIM)
    outs = _sc_gather(prefix, table_r)
    return tuple(
        o.reshape(2, _BATCH, _N_HEADS, _PRE_SEQ_LEN, _HEAD_DIM) for o in outs)


# speedup vs baseline: 2.3417x; 1.0686x over previous
import functools

import jax
import jax.numpy as jnp
from jax import lax
from jax.experimental import pallas as pl
from jax.experimental.pallas import tpu as pltpu
from jax.experimental.pallas import tpu_sc as plsc

_N_LAYERS = 24
_N_HEADS = 32
_HEAD_DIM = 64
_PRE_SEQ_LEN = 64
_BATCH = 4
_ROW_STRIDE = _N_LAYERS * 2 * _N_HEADS
_ROWS_PER_OUT = 2 * _BATCH * _N_HEADS * _PRE_SEQ_LEN
_NW = 32
_ROWS_PER_W = _ROWS_PER_OUT // _NW
_CHUNK = 128
_NCHUNK = _ROWS_PER_W // _CHUNK


def _body(prefix_hbm, table_hbm, *refs):
    outs = refs[:_N_LAYERS]
    pref_v, idx_v, buf_a, buf_b, gsem, ssem = refs[_N_LAYERS:]
    bufs = (buf_a, buf_b)

    wid = lax.axis_index("s") * 2 + lax.axis_index("c")
    kv = wid // 16
    b = (wid // 4) % 4
    h0 = (wid % 4) * 8
    woff = kv * 32 + h0 - 64
    base_out = wid * _ROWS_PER_W

    pltpu.sync_copy(prefix_hbm.at[b], pref_v)

    for c in range(_NCHUNK):
        for j in range(8):
            r0 = c * _CHUNK + j * 16
            keys = pref_v[pl.ds(r0 % _PRE_SEQ_LEN, 16)]
            idx_v[c, pl.ds(j * 16, 16)] = keys * _ROW_STRIDE + (woff + r0 // 64)

    for i in range(_N_LAYERS):
        out_i = outs[i]
        lb = bufs[i % 2]
        out_slice = out_i.at[pl.ds(base_out, _ROWS_PER_W)]

        if i >= 2:
            prev_out = outs[i - 2]
            pltpu.make_async_copy(
                lb, prev_out.at[pl.ds(base_out, _ROWS_PER_W)],
                ssem.at[i % 2]).wait()

        @pl.loop(0, _NCHUNK)
        def _fire(c):
            for j in range(8):
                sl = pl.ds(j * 16, 16)
                idx_v[c, sl] = idx_v[c, sl] + 64
            pltpu.async_copy(table_hbm.at[idx_v.at[c]],
                             lb.at[pl.ds(c * _CHUNK, _CHUNK)], gsem)

        pltpu.make_async_copy(out_slice, lb, gsem).wait()

        pltpu.async_copy(lb, out_slice, ssem.at[i % 2])

    for i in (_N_LAYERS - 2, _N_LAYERS - 1):
        pltpu.make_async_copy(
            bufs[i % 2], outs[i].at[pl.ds(base_out, _ROWS_PER_W)],
            ssem.at[i % 2]).wait()


@functools.partial(jax.jit, static_argnames=())
def _sc_gather(prefix, table_r):
    mesh = plsc.VectorSubcoreMesh(core_axis_name="c", subcore_axis_name="s")
    out_type = [jax.ShapeDtypeStruct((_ROWS_PER_OUT, _HEAD_DIM), jnp.float32)
                for _ in range(_N_LAYERS)]
    scratch = [
        pltpu.VMEM((_PRE_SEQ_LEN,), jnp.int32),
        pltpu.VMEM((_NCHUNK, _CHUNK), jnp.int32),
        pltpu.VMEM((_ROWS_PER_W, _HEAD_DIM), jnp.float32),
        pltpu.VMEM((_ROWS_PER_W, _HEAD_DIM), jnp.float32),
        pltpu.SemaphoreType.DMA,
        pltpu.SemaphoreType.DMA((2,)),
    ]
    return pl.kernel(
        _body, out_type=out_type, mesh=mesh, scratch_types=scratch,
        compiler_params=pltpu.CompilerParams(use_tc_tiling_on_sc=False),
    )(prefix, table_r)


def kernel(prefix, table):
    table_r = table.reshape(_PRE_SEQ_LEN * _ROW_STRIDE, _HEAD_DIM)
    outs = _sc_gather(prefix, table_r)
    return tuple(
        o.reshape(2, _BATCH, _N_HEADS, _PRE_SEQ_LEN, _HEAD_DIM) for o in outs)
